# baseline (speedup 1.0000x reference)
import functools

import jax
import jax.numpy as jnp
from jax import lax
from jax.experimental import pallas as pl
from jax.experimental.pallas import tpu as pltpu
from jax.experimental.pallas import tpu_sc as plsc

VOCAB = 1000
D = 1000
B = 1024
T = 200
NW = 32
BT = B // 128
NFRAG = T * BT
FPW = NFRAG // NW
DSLAB = 40
NSLAB = D // DSLAB
SLABW = DSLAB * VOCAB
NITER = NSLAB * FPW


def _body(tableT_hbm, idxT_hbm, out_hbm, idx_v, slab_v, frag_v, sem_o,
          sem_s):
    c_ax = lax.axis_index("c")
    s_ax = lax.axis_index("s")
    wid = s_ax * 2 + c_ax

    def stage_idx(k, carry):
        f = wid + NW * k
        pltpu.sync_copy(
            idxT_hbm.at[pl.ds(f * 128, 128)],
            idx_v.at[pl.ds(k * 128, 128)],
        )
        return carry

    lax.fori_loop(0, FPW, stage_idx, 0)

    def frag_dma(k, s, p):
        f = wid + NW * k
        t = f // BT
        bt = lax.rem(f, BT)
        return pltpu.make_async_copy(
            frag_v.at[p],
            out_hbm.at[t, pl.ds(s * DSLAB, DSLAB), pl.ds(bt * 128, 128)],
            sem_o.at[p],
        )

    def stage_dma(s, sb):
        return pltpu.make_async_copy(
            tableT_hbm.at[pl.ds(s * SLABW, SLABW)],
            slab_v.at[pl.ds(sb * SLABW, SLABW)],
            sem_s.at[sb],
        )

    stage_dma(0, 0).start()

    def body(i, carry):
        s = i // FPW
        k = lax.rem(i, FPW)
        p = lax.rem(i, 2)
        sb = lax.rem(s, 2)

        @pl.when(k == 0)
        def _():
            stage_dma(s, sb).wait()

            @pl.when(s + 1 < NSLAB)
            def _():
                stage_dma(s + 1, 1 - sb).start()

        @pl.when(i >= 2)
        def _():
            frag_dma(k, s, p).wait()

        off = sb * SLABW
        tok = [
            idx_v[pl.ds(k * 128 + 16 * j, 16)] + off for j in range(8)
        ]

        UN = 4

        def drow(h, iv):
            d = UN * h
            gots = []
            for u in range(UN):
                gots.append(
                    [plsc.load_gather(slab_v, [iv[j]]) for j in range(8)]
                )
                iv = tuple(v + VOCAB for v in iv)
            for u in range(UN):
                for j in range(8):
                    frag_v[p, d + u, pl.ds(16 * j, 16)] = gots[u][j]
            return iv

        lax.fori_loop(0, DSLAB // UN, drow, tuple(tok))
        frag_dma(k, s, p).start()
        return carry

    lax.fori_loop(0, NITER, body, 0)
    frag_dma(FPW - 1, NSLAB - 1, 0).wait()
    frag_dma(FPW - 1, NSLAB - 1, 1).wait()


@jax.jit
def kernel(idx, token_embedding):
    idxT_flat = idx.T.reshape(-1)
    tableT_flat = token_embedding.T.reshape(-1)
    mesh = plsc.VectorSubcoreMesh(core_axis_name="c", subcore_axis_name="s")
    out = pl.kernel(
        _body,
        out_type=jax.ShapeDtypeStruct((T, D, B), jnp.float32),
        mesh=mesh,
        scratch_types=[
            pltpu.VMEM((FPW * 128,), jnp.int32),
            pltpu.VMEM((2 * SLABW,), jnp.float32),
            pltpu.VMEM((2, DSLAB, 128), jnp.float32),
            pltpu.SemaphoreType.DMA((2,)),
            pltpu.SemaphoreType.DMA((2,)),
        ],
        compiler_params=pltpu.CompilerParams(needs_layout_passes=False),
    )(tableT_flat, idxT_flat)
    return out.transpose(2, 0, 1)


# speedup vs baseline: 1.0377x; 1.0377x over previous
import functools

import jax
import jax.numpy as jnp
from jax import lax
from jax.experimental import pallas as pl
from jax.experimental.pallas import tpu as pltpu
from jax.experimental.pallas import tpu_sc as plsc

VOCAB = 1000
D = 1000
B = 1024
T = 200
NW = 32
BT = B // 128
NFRAG = T * BT
FPW = NFRAG // NW
DSLAB = 40
NSLAB = D // DSLAB
SLABW = DSLAB * VOCAB
NITER = NSLAB * FPW


def _body(tableT_hbm, idxT_hbm, out_hbm, idx_v, slab_v, frag_v, sem_o,
          sem_s):
    c_ax = lax.axis_index("c")
    s_ax = lax.axis_index("s")
    wid = s_ax * 2 + c_ax

    def stage_idx(k, carry):
        f = wid + NW * k
        pltpu.sync_copy(
            idxT_hbm.at[pl.ds(f * 128, 128)],
            idx_v.at[pl.ds(k * 128, 128)],
        )
        return carry

    lax.fori_loop(0, FPW, stage_idx, 0)

    def frag_dma(k, s, p):
        f = wid + NW * k
        t = f // BT
        bt = lax.rem(f, BT)
        return pltpu.make_async_copy(
            frag_v.at[p],
            out_hbm.at[t, pl.ds(s * DSLAB, DSLAB), pl.ds(bt * 128, 128)],
            sem_o.at[p],
        )

    def stage_dma(s, sb):
        return pltpu.make_async_copy(
            tableT_hbm.at[pl.ds(s * SLABW, SLABW)],
            slab_v.at[pl.ds(sb * SLABW, SLABW)],
            sem_s.at[sb],
        )

    stage_dma(0, 0).start()

    def process(k, s, off, p):
        tok = [
            idx_v[pl.ds(k * 128 + 16 * j, 16)] + off for j in range(8)
        ]

        def drow(h, iv):
            d = 2 * h
            got = [plsc.load_gather(slab_v, [iv[j]]) for j in range(8)]
            iv2 = tuple(v + VOCAB for v in iv)
            got2 = [plsc.load_gather(slab_v, [iv2[j]]) for j in range(8)]
            for j in range(8):
                frag_v[p, d, pl.ds(16 * j, 16)] = got[j]
            for j in range(8):
                frag_v[p, d + 1, pl.ds(16 * j, 16)] = got2[j]
            return tuple(v + VOCAB for v in iv2)

        lax.fori_loop(0, DSLAB // 2, drow, tuple(tok))
        frag_dma(k, s, p).start()

    def body(ii, carry):
        i0 = 2 * ii
        s = i0 // FPW
        k0 = lax.rem(i0, FPW)
        sb = lax.rem(s, 2)

        @pl.when(k0 == 0)
        def _():
            stage_dma(s, sb).wait()

            @pl.when(s + 1 < NSLAB)
            def _():
                stage_dma(s + 1, 1 - sb).start()

        off = sb * SLABW

        @pl.when(ii >= 1)
        def _():
            frag_dma(k0, s, 0).wait()

        process(k0, s, off, 0)

        @pl.when(ii >= 1)
        def _():
            frag_dma(k0 + 1, s, 1).wait()

        process(k0 + 1, s, off, 1)
        return carry

    lax.fori_loop(0, NITER // 2, body, 0)
    frag_dma(FPW - 1, NSLAB - 1, 0).wait()
    frag_dma(FPW - 1, NSLAB - 1, 1).wait()


@jax.jit
def kernel(idx, token_embedding):
    idxT_flat = idx.T.reshape(-1)
    tableT_flat = token_embedding.T.reshape(-1)
    mesh = plsc.VectorSubcoreMesh(core_axis_name="c", subcore_axis_name="s")
    out = pl.kernel(
        _body,
        out_type=jax.ShapeDtypeStruct((T, D, B), jnp.float32),
        mesh=mesh,
        scratch_types=[
            pltpu.VMEM((FPW * 128,), jnp.int32),
            pltpu.VMEM((2 * SLABW,), jnp.float32),
            pltpu.VMEM((2, DSLAB, 128), jnp.float32),
            pltpu.SemaphoreType.DMA((2,)),
            pltpu.SemaphoreType.DMA((2,)),
        ],
        compiler_params=pltpu.CompilerParams(needs_layout_passes=False),
    )(tableT_flat, idxT_flat)
    return out.transpose(2, 0, 1)
